# baseline (speedup 1.0000x reference)
import functools

import jax
import jax.numpy as jnp
from jax import lax
from jax.experimental import pallas as pl
from jax.experimental.pallas import tpu as pltpu
from jax.experimental.pallas import tpu_sc as plsc

_CHUNK = 2000


def _zero_f32(ref, n, lanes):
    zeros = jnp.zeros((lanes,), jnp.float32)

    @plsc.parallel_loop(0, n // lanes, unroll=8)
    def _(i):
        ref[pl.ds(i * lanes, lanes)] = zeros


def _make_deg_kernel(n_pad, e_pad, nw, nc, lanes):
    per_tile = e_pad // nw
    n_chunks = per_tile // _CHUNK
    assert n_chunks % 2 == 0
    mesh = plsc.VectorSubcoreMesh(core_axis_name="c", subcore_axis_name="s")

    @functools.partial(
        pl.kernel,
        mesh=mesh,
        compiler_params=pltpu.CompilerParams(needs_layout_passes=False),
        out_type=jax.ShapeDtypeStruct((nw, n_pad), jnp.float32),
        scratch_types=[
            pltpu.VMEM((_CHUNK,), jnp.int32),
            pltpu.VMEM((_CHUNK,), jnp.int32),
            pltpu.VMEM((n_pad,), jnp.float32),
            pltpu.SemaphoreType.DMA,
            pltpu.SemaphoreType.DMA,
        ],
    )
    def deg_kernel(edges_hbm, out_hbm, dst_v0, dst_v1, acc_v, sem0, sem1):
        wid = lax.axis_index("s") * nc + lax.axis_index("c")
        base = wid * per_tile
        sems = (sem0, sem1)
        dst_bufs = (dst_v0, dst_v1)

        def copy(ci, b):
            return pltpu.make_async_copy(
                edges_hbm.at[pl.ds(e_pad + base + ci * _CHUNK, _CHUNK)],
                dst_bufs[b],
                sems[b],
            )

        copy(0, 0).start()
        copy(1, 1).start()
        _zero_f32(acc_v, n_pad, lanes)
        ones = jnp.ones((lanes,), jnp.float32)

        def pair_body(p, _):
            for b in (0, 1):
                ci = 2 * p + b
                copy(ci, b).wait()

                dvb = dst_bufs[b]

                @plsc.parallel_loop(0, _CHUNK // lanes, unroll=8)
                def _(i):
                    d = dvb[pl.ds(i * lanes, lanes)]
                    plsc.addupdate_scatter(acc_v, [d], ones)

                @pl.when(ci + 2 < n_chunks)
                def _():
                    copy(ci + 2, b).start()

            return 0

        lax.fori_loop(0, n_chunks // 2, pair_body, 0)
        pltpu.sync_copy(acc_v, out_hbm.at[wid])

    return deg_kernel


def _make_agg_kernel(n_pad, e_pad, nw, nc, lanes, nf):
    groups = nw // nf
    per_tile = e_pad // groups
    n_chunks = per_tile // _CHUNK
    assert n_chunks % 2 == 0
    mesh = plsc.VectorSubcoreMesh(core_axis_name="c", subcore_axis_name="s")

    @functools.partial(
        pl.kernel,
        mesh=mesh,
        compiler_params=pltpu.CompilerParams(needs_layout_passes=False),
        out_type=jax.ShapeDtypeStruct((groups, nf, n_pad), jnp.float32),
        scratch_types=[
            pltpu.VMEM((_CHUNK,), jnp.int32),
            pltpu.VMEM((_CHUNK,), jnp.int32),
            pltpu.VMEM((_CHUNK,), jnp.int32),
            pltpu.VMEM((_CHUNK,), jnp.int32),
            pltpu.VMEM((n_pad,), jnp.float32),
            pltpu.VMEM((n_pad,), jnp.float32),
            pltpu.SemaphoreType.DMA,
            pltpu.SemaphoreType.DMA,
            pltpu.SemaphoreType.DMA,
        ],
    )
    def agg_kernel(
        edges_hbm, y_hbm, out_hbm,
        src_v0, src_v1, dst_v0, dst_v1, y_v, acc_v, sem0, sem1, ysem,
    ):
        wid = lax.axis_index("s") * nc + lax.axis_index("c")
        f = wid % nf
        g = wid // nf
        base = g * per_tile
        sems = (sem0, sem1)
        src_bufs = (src_v0, src_v1)
        dst_bufs = (dst_v0, dst_v1)

        def copy(ci, b):
            off = base + ci * _CHUNK
            return (
                pltpu.make_async_copy(
                    edges_hbm.at[pl.ds(off, _CHUNK)], src_bufs[b], sems[b]
                ),
                pltpu.make_async_copy(
                    edges_hbm.at[pl.ds(e_pad + off, _CHUNK)], dst_bufs[b], sems[b]
                ),
            )

        def start(ci, b):
            c0, c1 = copy(ci, b)
            c0.start()
            c1.start()

        def wait(ci, b):
            c0, c1 = copy(ci, b)
            c0.wait()
            c1.wait()

        ycopy = pltpu.make_async_copy(y_hbm.at[f], y_v, ysem)
        ycopy.start()
        start(0, 0)
        start(1, 1)
        _zero_f32(acc_v, n_pad, lanes)
        ycopy.wait()

        def pair_body(p, _):
            for b in (0, 1):
                ci = 2 * p + b
                wait(ci, b)

                svb, dvb = src_bufs[b], dst_bufs[b]

                @plsc.parallel_loop(0, _CHUNK // lanes, unroll=8)
                def _(i):
                    s = svb[pl.ds(i * lanes, lanes)]
                    d = dvb[pl.ds(i * lanes, lanes)]
                    v = plsc.load_gather(y_v, [s])
                    plsc.addupdate_scatter(acc_v, [d], v)

                @pl.when(ci + 2 < n_chunks)
                def _():
                    start(ci + 2, b)

            return 0

        lax.fori_loop(0, n_chunks // 2, pair_body, 0)
        pltpu.sync_copy(acc_v, out_hbm.at[g, f])

    return agg_kernel


def _tc0_body(deg_ref, x_ref, W1_ref, dinv_ref, z_ref, y_ref):
    deg = jnp.sum(deg_ref[...], axis=0, keepdims=True) + 1.0
    dinv = lax.rsqrt(deg)
    dinv_ref[...] = dinv
    n = z_ref.shape[1]
    zn = jnp.dot(x_ref[...], W1_ref[...], preferred_element_type=jnp.float32)
    z = zn.T
    if z.shape[1] != n:
        z = jnp.pad(z, ((0, 0), (0, n - z.shape[1])))
    z_ref[...] = z
    y_ref[...] = z * dinv


def _tc_mid_body(part_ref, z_ref, dinv_ref, b_ref, WT_ref, z2_ref, y2_ref):
    dinv = dinv_ref[...]
    agg = jnp.sum(part_ref[...], axis=0)
    h = jnp.tanh(dinv * agg + dinv * dinv * z_ref[...] + b_ref[...])
    z2 = jnp.dot(WT_ref[...], h, preferred_element_type=jnp.float32)
    z2_ref[...] = z2
    y2_ref[...] = z2 * dinv


def _tc_final_body(part_ref, z_ref, dinv_ref, b_ref, WcT_ref, bc_ref, h_ref, out_ref):
    dinv = dinv_ref[...]
    agg = jnp.sum(part_ref[...], axis=0)
    h = jnp.tanh(dinv * agg + dinv * dinv * z_ref[...] + b_ref[...])
    h_ref[...] = h.T
    out_ref[...] = (
        jnp.dot(WcT_ref[...], h, preferred_element_type=jnp.float32) + bc_ref[...]
    ).T


def kernel(x, edge_index, W1, b1, W2, b2, W3, b3, Wc, bc):
    N = x.shape[0]
    E = edge_index.shape[1]
    f1 = W1.shape[1]
    f2 = W2.shape[1]
    f3 = W3.shape[1]

    info = plsc.get_sparse_core_info()
    nc, ns, lanes = info.num_cores, info.num_subcores, info.num_lanes
    nw = nc * ns

    e_quant = nw * _CHUNK * 2
    e_pad = ((E + e_quant - 1) // e_quant) * e_quant
    extra = 0 if e_pad == E else 1
    n_pad = ((N + extra + lanes - 1) // lanes) * lanes

    edges = edge_index.astype(jnp.int32)
    if e_pad != E:
        pad = jnp.full((2, e_pad - E), N, dtype=jnp.int32)
        edges = jnp.concatenate([edges, pad], axis=1)
    edges = edges.reshape(-1)

    deg_kernel = _make_deg_kernel(n_pad, e_pad, nw, nc, lanes)
    deg_part = deg_kernel(edges)

    dinv, z1, y1 = pl.pallas_call(
        _tc0_body,
        out_shape=[
            jax.ShapeDtypeStruct((1, n_pad), jnp.float32),
            jax.ShapeDtypeStruct((f1, n_pad), jnp.float32),
            jax.ShapeDtypeStruct((f1, n_pad), jnp.float32),
        ],
    )(deg_part, x, W1)

    agg1 = _make_agg_kernel(n_pad, e_pad, nw, nc, lanes, f1)
    part1 = agg1(edges, y1)

    z2, y2 = pl.pallas_call(
        _tc_mid_body,
        out_shape=[
            jax.ShapeDtypeStruct((f2, n_pad), jnp.float32),
            jax.ShapeDtypeStruct((f2, n_pad), jnp.float32),
        ],
    )(part1, z1, dinv, b1.reshape(f1, 1), W2.T)

    agg2 = agg1 if f2 == f1 else _make_agg_kernel(n_pad, e_pad, nw, nc, lanes, f2)
    part2 = agg2(edges, y2)

    z3, y3 = pl.pallas_call(
        _tc_mid_body,
        out_shape=[
            jax.ShapeDtypeStruct((f3, n_pad), jnp.float32),
            jax.ShapeDtypeStruct((f3, n_pad), jnp.float32),
        ],
    )(part2, z2, dinv, b2.reshape(f2, 1), W3.T)

    agg3 = _make_agg_kernel(n_pad, e_pad, nw, nc, lanes, f3)
    part3 = agg3(edges, y3)

    h_full, out_full = pl.pallas_call(
        _tc_final_body,
        out_shape=[
            jax.ShapeDtypeStruct((n_pad, f3), jnp.float32),
            jax.ShapeDtypeStruct((n_pad, Wc.shape[1]), jnp.float32),
        ],
    )(part3, z3, dinv, b3.reshape(f3, 1), Wc.T, bc.reshape(Wc.shape[1], 1))

    out = out_full[:N]
    h = h_full[:N]
    return (out, h)


# speedup vs baseline: 1.1520x; 1.1520x over previous
import functools

import jax
import jax.numpy as jnp
from jax import lax
from jax.experimental import pallas as pl
from jax.experimental.pallas import tpu as pltpu
from jax.experimental.pallas import tpu_sc as plsc

_CHUNK = 2000


def _zero_f32(ref, n, lanes):
    zeros = jnp.zeros((lanes,), jnp.float32)

    @plsc.parallel_loop(0, n // lanes, unroll=8)
    def _(i):
        ref[pl.ds(i * lanes, lanes)] = zeros


def _make_deg_kernel(n_pad, e_pad, nw, nc, lanes):
    per_tile = e_pad // nw
    n_chunks = per_tile // _CHUNK
    assert n_chunks % 2 == 0
    mesh = plsc.VectorSubcoreMesh(core_axis_name="c", subcore_axis_name="s")

    @functools.partial(
        pl.kernel,
        mesh=mesh,
        compiler_params=pltpu.CompilerParams(needs_layout_passes=False),
        out_type=jax.ShapeDtypeStruct((nw, n_pad), jnp.float32),
        scratch_types=[
            pltpu.VMEM((_CHUNK,), jnp.int32),
            pltpu.VMEM((_CHUNK,), jnp.int32),
            pltpu.VMEM((n_pad,), jnp.float32),
            pltpu.SemaphoreType.DMA,
            pltpu.SemaphoreType.DMA,
        ],
    )
    def deg_kernel(edges_hbm, out_hbm, dst_v0, dst_v1, acc_v, sem0, sem1):
        wid = lax.axis_index("s") * nc + lax.axis_index("c")
        base = wid * per_tile
        sems = (sem0, sem1)
        dst_bufs = (dst_v0, dst_v1)

        def copy(ci, b):
            return pltpu.make_async_copy(
                edges_hbm.at[pl.ds(e_pad + base + ci * _CHUNK, _CHUNK)],
                dst_bufs[b],
                sems[b],
            )

        copy(0, 0).start()
        copy(1, 1).start()
        _zero_f32(acc_v, n_pad, lanes)
        ones = jnp.ones((lanes,), jnp.float32)

        def pair_body(p, _):
            for b in (0, 1):
                ci = 2 * p + b
                copy(ci, b).wait()

                dvb = dst_bufs[b]

                @plsc.parallel_loop(0, _CHUNK // lanes, unroll=8)
                def _(i):
                    d = dvb[pl.ds(i * lanes, lanes)]
                    plsc.addupdate_scatter(acc_v, [d], ones)

                @pl.when(ci + 2 < n_chunks)
                def _():
                    copy(ci + 2, b).start()

            return 0

        lax.fori_loop(0, n_chunks // 2, pair_body, 0)
        pltpu.sync_copy(acc_v, out_hbm.at[wid])

    return deg_kernel


def _make_agg_kernel(n_pad, e_pad, nw, nc, lanes, nf):
    groups = nw // nf
    per_tile = e_pad // groups
    n_chunks = per_tile // _CHUNK
    assert n_chunks % 2 == 0
    mesh = plsc.VectorSubcoreMesh(core_axis_name="c", subcore_axis_name="s")

    @functools.partial(
        pl.kernel,
        mesh=mesh,
        compiler_params=pltpu.CompilerParams(needs_layout_passes=False),
        out_type=jax.ShapeDtypeStruct((groups, nf, n_pad), jnp.float32),
        scratch_types=[
            pltpu.VMEM((_CHUNK,), jnp.int32),
            pltpu.VMEM((_CHUNK,), jnp.int32),
            pltpu.VMEM((_CHUNK,), jnp.int32),
            pltpu.VMEM((_CHUNK,), jnp.int32),
            pltpu.VMEM((n_pad,), jnp.float32),
            pltpu.VMEM((n_pad,), jnp.float32),
            pltpu.SemaphoreType.DMA,
            pltpu.SemaphoreType.DMA,
            pltpu.SemaphoreType.DMA,
        ],
    )
    def agg_kernel(
        edges_hbm, y_hbm, out_hbm,
        src_v0, src_v1, dst_v0, dst_v1, y_v, acc_v, sem0, sem1, ysem,
    ):
        wid = lax.axis_index("s") * nc + lax.axis_index("c")
        f = wid % nf
        g = wid // nf
        base = g * per_tile
        sems = (sem0, sem1)
        src_bufs = (src_v0, src_v1)
        dst_bufs = (dst_v0, dst_v1)

        def copy(ci, b):
            off = base + ci * _CHUNK
            return (
                pltpu.make_async_copy(
                    edges_hbm.at[pl.ds(off, _CHUNK)], src_bufs[b], sems[b]
                ),
                pltpu.make_async_copy(
                    edges_hbm.at[pl.ds(e_pad + off, _CHUNK)], dst_bufs[b], sems[b]
                ),
            )

        def start(ci, b):
            c0, c1 = copy(ci, b)
            c0.start()
            c1.start()

        def wait(ci, b):
            c0, c1 = copy(ci, b)
            c0.wait()
            c1.wait()

        ycopy = pltpu.make_async_copy(y_hbm.at[f], y_v, ysem)
        ycopy.start()
        start(0, 0)
        start(1, 1)
        _zero_f32(acc_v, n_pad, lanes)
        ycopy.wait()

        def pair_body(p, _):
            for b in (0, 1):
                ci = 2 * p + b
                wait(ci, b)

                svb, dvb = src_bufs[b], dst_bufs[b]

                @plsc.parallel_loop(0, _CHUNK // lanes, unroll=8)
                def _(i):
                    s = svb[pl.ds(i * lanes, lanes)]
                    d = dvb[pl.ds(i * lanes, lanes)]
                    v = plsc.load_gather(y_v, [s])
                    plsc.addupdate_scatter(acc_v, [d], v)

                @pl.when(ci + 2 < n_chunks)
                def _():
                    start(ci + 2, b)

            return 0

        lax.fori_loop(0, n_chunks // 2, pair_body, 0)
        pltpu.sync_copy(acc_v, out_hbm.at[g, f])

    return agg_kernel


def _tc0_body(deg_ref, xT_ref, W1T_ref, dinv_ref, z_ref, y_ref):
    deg = jnp.sum(deg_ref[...], axis=0, keepdims=True) + 1.0
    dinv = lax.rsqrt(deg)
    dinv_ref[...] = dinv
    n = z_ref.shape[1]
    z = jnp.dot(W1T_ref[...], xT_ref[...], preferred_element_type=jnp.float32)
    if z.shape[1] != n:
        z = jnp.pad(z, ((0, 0), (0, n - z.shape[1])))
    z_ref[...] = z
    y_ref[...] = z * dinv


def _tc_mid_body(part_ref, z_ref, dinv_ref, b_ref, WT_ref, z2_ref, y2_ref):
    dinv = dinv_ref[...]
    agg = jnp.sum(part_ref[...], axis=0)
    h = jnp.tanh(dinv * agg + dinv * dinv * z_ref[...] + b_ref[...])
    z2 = jnp.dot(WT_ref[...], h, preferred_element_type=jnp.float32)
    z2_ref[...] = z2
    y2_ref[...] = z2 * dinv


def _tc_final_body(part_ref, z_ref, dinv_ref, b_ref, WcT_ref, bc_ref, h_ref, out_ref):
    dinv = dinv_ref[...]
    agg = jnp.sum(part_ref[...], axis=0)
    h = jnp.tanh(dinv * agg + dinv * dinv * z_ref[...] + b_ref[...])
    h_ref[...] = h
    out_ref[...] = (
        jnp.dot(WcT_ref[...], h, preferred_element_type=jnp.float32) + bc_ref[...]
    )


def kernel(x, edge_index, W1, b1, W2, b2, W3, b3, Wc, bc):
    N = x.shape[0]
    E = edge_index.shape[1]
    f1 = W1.shape[1]
    f2 = W2.shape[1]
    f3 = W3.shape[1]

    info = plsc.get_sparse_core_info()
    nc, ns, lanes = info.num_cores, info.num_subcores, info.num_lanes
    nw = nc * ns

    e_quant = nw * _CHUNK * 2
    e_pad = ((E + e_quant - 1) // e_quant) * e_quant
    extra = 0 if e_pad == E else 1
    n_pad = ((N + extra + lanes - 1) // lanes) * lanes

    edges = edge_index.astype(jnp.int32)
    if e_pad != E:
        pad = jnp.full((2, e_pad - E), N, dtype=jnp.int32)
        edges = jnp.concatenate([edges, pad], axis=1)
    edges = edges.reshape(-1)

    deg_kernel = _make_deg_kernel(n_pad, e_pad, nw, nc, lanes)
    deg_part = deg_kernel(edges)

    dinv, z1, y1 = pl.pallas_call(
        _tc0_body,
        out_shape=[
            jax.ShapeDtypeStruct((1, n_pad), jnp.float32),
            jax.ShapeDtypeStruct((f1, n_pad), jnp.float32),
            jax.ShapeDtypeStruct((f1, n_pad), jnp.float32),
        ],
    )(deg_part, x.T, W1.T)

    agg1 = _make_agg_kernel(n_pad, e_pad, nw, nc, lanes, f1)
    part1 = agg1(edges, y1)

    z2, y2 = pl.pallas_call(
        _tc_mid_body,
        out_shape=[
            jax.ShapeDtypeStruct((f2, n_pad), jnp.float32),
            jax.ShapeDtypeStruct((f2, n_pad), jnp.float32),
        ],
    )(part1, z1, dinv, b1.reshape(f1, 1), W2.T)

    agg2 = agg1 if f2 == f1 else _make_agg_kernel(n_pad, e_pad, nw, nc, lanes, f2)
    part2 = agg2(edges, y2)

    z3, y3 = pl.pallas_call(
        _tc_mid_body,
        out_shape=[
            jax.ShapeDtypeStruct((f3, n_pad), jnp.float32),
            jax.ShapeDtypeStruct((f3, n_pad), jnp.float32),
        ],
    )(part2, z2, dinv, b2.reshape(f2, 1), W3.T)

    agg3 = _make_agg_kernel(n_pad, e_pad, nw, nc, lanes, f3)
    part3 = agg3(edges, y3)

    hT, outT = pl.pallas_call(
        _tc_final_body,
        out_shape=[
            jax.ShapeDtypeStruct((f3, n_pad), jnp.float32),
            jax.ShapeDtypeStruct((Wc.shape[1], n_pad), jnp.float32),
        ],
    )(part3, z3, dinv, b3.reshape(f3, 1), Wc.T, bc.reshape(Wc.shape[1], 1))

    out = outT[:, :N].T
    h = hT[:, :N].T
    return (out, h)
